# baseline (speedup 1.0000x reference)
import jax
import jax.numpy as jnp
from jax import lax
from jax.experimental import pallas as pl
from jax.experimental.pallas import tpu as pltpu
from jax.experimental.pallas import tpu_sc as plsc

NC = 2
NS = 16
NW = NC * NS
L = 16

BATCH = 16384
EMB_DIM = 16
B_PER_W = BATCH // NW
N_BLOCKS = B_PER_W // L
IDX_MINOR = 128
N_STREAMS = B_PER_W // IDX_MINOR


def _gmf_body(x_hbm, uid_hbm, iid_hbm, w_hbm, out_hbm,
              xv, uidx, iidx, urows, irows, wv, out_v, sem):
    wid = lax.axis_index("s") * NC + lax.axis_index("c")
    base = wid * B_PER_W

    pltpu.sync_copy(x_hbm.at[pl.ds(base, B_PER_W)], xv)
    pltpu.sync_copy(w_hbm, wv)

    iota = lax.iota(jnp.int32, L)
    col0 = jnp.zeros((L,), jnp.int32)
    col1 = jnp.ones((L,), jnp.int32)
    for m in range(N_BLOCKS):
        rows = iota + (m * L)
        u16 = plsc.load_gather(xv, [rows, col0])
        i16 = plsc.load_gather(xv, [rows, col1])
        uidx[m // 8, pl.ds((m % 8) * L, L)] = u16
        iidx[m // 8, pl.ds((m % 8) * L, L)] = i16

    copies = []
    for k in range(N_STREAMS):
        copies.append(pltpu.async_copy(
            uid_hbm.at[uidx.at[k]], urows.at[pl.ds(k * IDX_MINOR, IDX_MINOR)],
            sem))
        copies.append(pltpu.async_copy(
            iid_hbm.at[iidx.at[k]], irows.at[pl.ds(k * IDX_MINOR, IDX_MINOR)],
            sem))
    for c in copies:
        c.wait()

    wvals = [wv[d, :] for d in range(EMB_DIM)]

    def block(j, carry):
        rows = iota + j * L
        acc = jnp.zeros((L,), jnp.float32)
        for d in range(EMB_DIM):
            cd = jnp.full((L,), d, jnp.int32)
            u = plsc.load_gather(urows, [rows, cd])
            i = plsc.load_gather(irows, [rows, cd])
            acc = acc + u * i * wvals[d]
        out_v[pl.ds(j * L, L)] = acc
        return carry

    lax.fori_loop(0, N_BLOCKS, block, 0)

    pltpu.sync_copy(out_v, out_hbm.at[pl.ds(base, B_PER_W)])


def _make_kernel():
    mesh = plsc.VectorSubcoreMesh(core_axis_name="c", subcore_axis_name="s")
    return pl.kernel(
        _gmf_body,
        out_type=jax.ShapeDtypeStruct((BATCH,), jnp.float32),
        mesh=mesh,
        scratch_types=[
            pltpu.VMEM((B_PER_W, 2), jnp.int32),
            pltpu.VMEM((N_STREAMS, IDX_MINOR), jnp.int32),
            pltpu.VMEM((N_STREAMS, IDX_MINOR), jnp.int32),
            pltpu.VMEM((B_PER_W, EMB_DIM), jnp.float32),
            pltpu.VMEM((B_PER_W, EMB_DIM), jnp.float32),
            pltpu.VMEM((EMB_DIM, L), jnp.float32),
            pltpu.VMEM((B_PER_W,), jnp.float32),
            pltpu.SemaphoreType.DMA,
        ],
        compiler_params=pltpu.CompilerParams(
            needs_layout_passes=False, use_tc_tiling_on_sc=False),
    )


_gmf = _make_kernel()


def kernel(x, stage, uid_table, iid_table, W):
    del stage
    wmat = jnp.broadcast_to(W.reshape(EMB_DIM, 1), (EMB_DIM, L))
    return _gmf(x, uid_table, iid_table, wmat)


# speedup vs baseline: 1.1360x; 1.1360x over previous
import jax
import jax.numpy as jnp
from jax import lax
from jax.experimental import pallas as pl
from jax.experimental.pallas import tpu as pltpu
from jax.experimental.pallas import tpu_sc as plsc

NC = 2
NS = 16
NW = NC * NS
L = 16

BATCH = 16384
EMB_DIM = 16
VOCAB = 1000000
TCOLS = 7813
SLAB_TC = 244
SLAB_V = SLAB_TC * 128
TAIL_LO = 7808 * 128
TAIL_N = VOCAB - TAIL_LO
TAIL_PAD = 640
BLK_TC = 16
BLK_V = BLK_TC * 128
N_BLK = 16
CLAMP_C0 = (VOCAB - BLK_V) // 128
CAP = 768
B_PER_W = BATCH // NW
N_CHUNK = 4
XCH = BATCH // N_CHUNK
SCHUNK = 64
PADW = 128
H_PER_W = B_PER_W // 2


def _p1_body(x_hbm, ut_hbm, it_hbm, utl_hbm, itl_hbm, gu_hbm, gi_hbm,
             xs, blk0, blk1, tailb, ulv, ulb, ilv, ilb,
             ust, ist, ufl, ifl, cbuf, pad0, pad1,
             sem0, sem1, ssem0, ssem1):
    wid = lax.axis_index("s") * NC + lax.axis_index("c")
    s0 = wid * SLAB_TC
    lo = s0 * 128
    hi = lo + SLAB_V
    is0 = wid == 0

    iota = lax.iota(jnp.int32, L)
    trash = jnp.full((L,), BATCH, jnp.int32)

    for q in range(CAP // L):
        ufl[pl.ds(q * L, L)] = trash
        ifl[pl.ds(q * L, L)] = trash

    ucnt = jnp.zeros((), jnp.int32)
    icnt = jnp.zeros((), jnp.int32)
    for ch in range(N_CHUNK):
        pltpu.sync_copy(x_hbm.at[:, pl.ds(ch * XCH, XCH)], xs)

        def scan_body(t, carry, _ch=ch):
            uc, ic = carry
            rows = iota + t * L
            gb = rows + _ch * XCH
            vu = xs[0, pl.ds(t * L, L)]
            vi = xs[1, pl.ds(t * L, L)]
            mu = ((vu >= lo) & (vu < hi)) | (is0 & (vu >= TAIL_LO))
            mi = ((vi >= lo) & (vi < hi)) | (is0 & (vi >= TAIL_LO))
            plsc.store_compressed(ulv.at[pl.ds(uc, L)], vu, mask=mu)
            plsc.store_compressed(ulb.at[pl.ds(uc, L)], gb, mask=mu)
            plsc.store_compressed(ilv.at[pl.ds(ic, L)], vi, mask=mi)
            plsc.store_compressed(ilb.at[pl.ds(ic, L)], gb, mask=mi)
            uc = jnp.minimum(uc + jnp.sum(mu.astype(jnp.int32)), CAP)
            ic = jnp.minimum(ic + jnp.sum(mi.astype(jnp.int32)), CAP)
            return uc, ic

        ucnt, icnt = lax.fori_loop(0, XCH // L, scan_body, (ucnt, icnt))

    def process(tab_hbm, tl_hbm, lv, lb, cnt, stage, flat):
        nch = (cnt + (L - 1)) // L
        pltpu.sync_copy(tl_hbm, tailb)

        def start(j, buf, sem):
            c0 = jnp.minimum(s0 + j * BLK_TC, CLAMP_C0)
            c0w = pl.multiple_of(c0 * 128, 128)
            return c0, pltpu.async_copy(
                tab_hbm.at[:, pl.ds(c0w, BLK_V)], buf, sem)

        def run_step(buf, blo, bhi, rel0, scnt):
            def eb(t, sc, _buf=buf, _blo=blo, _bhi=bhi, _rel0=rel0):
                hv = lv[pl.ds(t * L, L)]
                hb = lb[pl.ds(t * L, L)]
                valid = ((iota + t * L) < cnt) & (hv >= _blo) & (hv < _bhi)
                hvr = jnp.where(valid, hv - _rel0, 0)
                nhit = jnp.sum(valid.astype(jnp.int32))
                plsc.store_compressed(flat.at[pl.ds(sc, L)], hb, mask=valid)
                plsc.store_compressed(cbuf.at[pl.ds(0, L)], hvr, mask=valid)
                hvc = cbuf[pl.ds(0, L)]

                def inner(q, sc2):
                    csp = hvc.at[jnp.full((L,), q, jnp.int32)].get(
                        mode="promise_in_bounds")
                    col = plsc.load_gather(_buf, [iota, csp])
                    stage[pl.ds(sc2 * EMB_DIM, EMB_DIM)] = col
                    return sc2 + 1

                return lax.fori_loop(0, nhit, inner, sc)

            return lax.fori_loop(0, nch, eb, scnt)

        scnt = jnp.zeros((), jnp.int32)
        c0_cur, cp_cur = start(0, blk0, sem0)
        for j in range(N_BLK):
            buf, nbuf = (blk0, blk1) if j % 2 == 0 else (blk1, blk0)
            nsem = sem1 if j % 2 == 0 else sem0
            if j + 1 < N_BLK:
                c0_nxt, cp_nxt = start(j + 1, nbuf, nsem)
            cp_cur.wait()
            blo = (s0 + j * BLK_TC) * 128
            scnt = run_step(buf, blo, blo + BLK_V, c0_cur * 128, scnt)
            if j + 1 < N_BLK:
                c0_cur, cp_cur = c0_nxt, cp_nxt
        run_step(tailb, TAIL_LO, VOCAB, TAIL_LO, scnt)

    process(ut_hbm, utl_hbm, ulv, ulb, ucnt, ust, ufl)
    process(it_hbm, itl_hbm, ilv, ilb, icnt, ist, ifl)

    def scatter_pass(stage, flat, dst_hbm):
        pending = [None, None]
        for c in range(CAP // SCHUNK):
            b = c % 2
            pb = pad0 if b == 0 else pad1
            psem = ssem0 if b == 0 else ssem1
            if pending[b] is not None:
                for cp in pending[b]:
                    cp.wait()
            def widen(r, carry, _pb=pb, _c=c):
                _pb[r, pl.ds(0, EMB_DIM)] = stage[
                    pl.ds((_c * SCHUNK + r) * EMB_DIM, EMB_DIM)]
                return carry

            lax.fori_loop(0, SCHUNK, widen, 0)
            pending[b] = [
                pltpu.async_copy(
                    pb.at[pl.ds(k * L, L), :],
                    dst_hbm.at[flat[pl.ds(c * SCHUNK + k * L, L)]], psem)
                for k in range(SCHUNK // L)]
        for p in pending:
            if p is not None:
                for cp in p:
                    cp.wait()

    scatter_pass(ust, ufl, gu_hbm)
    scatter_pass(ist, ifl, gi_hbm)


def _p2_body(gu_hbm, gi_hbm, w_hbm, out_hbm, gub, gib, wv, outv):
    wid = lax.axis_index("s") * NC + lax.axis_index("c")
    base = wid * B_PER_W
    pltpu.sync_copy(w_hbm, wv)
    iota = lax.iota(jnp.int32, L)
    wvals = [wv[d, pl.ds(0, L)] for d in range(EMB_DIM)]

    for h in range(2):
        hbase = base + h * H_PER_W
        pltpu.sync_copy(gu_hbm.at[pl.ds(hbase, H_PER_W), :], gub)
        pltpu.sync_copy(gi_hbm.at[pl.ds(hbase, H_PER_W), :], gib)

        def block(j, carry, _h=h):
            rows = iota + j * L
            acc = jnp.zeros((L,), jnp.float32)
            for d in range(EMB_DIM):
                cd = jnp.full((L,), d, jnp.int32)
                u = plsc.load_gather(gub, [rows, cd])
                i = plsc.load_gather(gib, [rows, cd])
                acc = acc + u * i * wvals[d]
            outv[pl.ds(_h * H_PER_W + j * L, L)] = acc
            return carry

        lax.fori_loop(0, H_PER_W // L, block, 0)
    pltpu.sync_copy(outv, out_hbm.at[pl.ds(base, B_PER_W)])


def _make_kernels():
    mesh = plsc.VectorSubcoreMesh(core_axis_name="c", subcore_axis_name="s")
    params = pltpu.CompilerParams(
        needs_layout_passes=False, use_tc_tiling_on_sc=True)
    p1 = pl.kernel(
        _p1_body,
        out_type=(jax.ShapeDtypeStruct((BATCH + 1, PADW), jnp.float32),
                  jax.ShapeDtypeStruct((BATCH + 1, PADW), jnp.float32)),
        mesh=mesh,
        scratch_types=[
            pltpu.VMEM((2, XCH), jnp.int32),
            pltpu.VMEM((EMB_DIM, BLK_V), jnp.float32),
            pltpu.VMEM((EMB_DIM, BLK_V), jnp.float32),
            pltpu.VMEM((EMB_DIM, TAIL_PAD), jnp.float32),
            pltpu.VMEM((CAP + L,), jnp.int32),
            pltpu.VMEM((CAP + L,), jnp.int32),
            pltpu.VMEM((CAP + L,), jnp.int32),
            pltpu.VMEM((CAP + L,), jnp.int32),
            pltpu.VMEM((CAP * EMB_DIM,), jnp.float32),
            pltpu.VMEM((CAP * EMB_DIM,), jnp.float32),
            pltpu.VMEM((CAP,), jnp.int32),
            pltpu.VMEM((CAP,), jnp.int32),
            pltpu.VMEM((L,), jnp.int32),
            pltpu.VMEM((SCHUNK, PADW), jnp.float32),
            pltpu.VMEM((SCHUNK, PADW), jnp.float32),
            pltpu.SemaphoreType.DMA,
            pltpu.SemaphoreType.DMA,
            pltpu.SemaphoreType.DMA,
            pltpu.SemaphoreType.DMA,
        ],
        compiler_params=params,
    )
    p2 = pl.kernel(
        _p2_body,
        out_type=jax.ShapeDtypeStruct((BATCH,), jnp.float32),
        mesh=mesh,
        scratch_types=[
            pltpu.VMEM((H_PER_W, PADW), jnp.float32),
            pltpu.VMEM((H_PER_W, PADW), jnp.float32),
            pltpu.VMEM((EMB_DIM, PADW), jnp.float32),
            pltpu.VMEM((B_PER_W,), jnp.float32),
        ],
        compiler_params=pltpu.CompilerParams(
            needs_layout_passes=False, use_tc_tiling_on_sc=False),
    )
    return p1, p2


_p1, _p2 = _make_kernels()


def kernel(x, stage, uid_table, iid_table, W):
    del stage
    wmat = jnp.broadcast_to(W.reshape(EMB_DIM, 1), (EMB_DIM, PADW))
    ut_tail = jnp.pad(uid_table[TAIL_LO:], ((0, TAIL_PAD - TAIL_N), (0, 0))).T
    it_tail = jnp.pad(iid_table[TAIL_LO:], ((0, TAIL_PAD - TAIL_N), (0, 0))).T
    gu, gi = _p1(x.T, uid_table.T, iid_table.T, ut_tail, it_tail)
    return _p2(gu, gi, wmat)
